# baseline (speedup 1.0000x reference)
import functools

import jax
import jax.numpy as jnp
from jax import lax
from jax.experimental import pallas as pl
from jax.experimental.pallas import tpu as pltpu
from jax.experimental.pallas import tpu_sc as plsc

EPS = 1e-12

NUM_CORES = 2
NUM_SUBCORES = 16
GATHER_CHUNK = 32
NBUF = 4
AHEAD = 3


def _sc_gather(table, idx):
    b, = idx.shape
    _, d = table.shape
    n_workers = NUM_CORES * NUM_SUBCORES
    b_per_w = b // n_workers
    n_chunks = b_per_w // GATHER_CHUNK
    mesh = plsc.VectorSubcoreMesh(
        core_axis_name="c", subcore_axis_name="s", num_cores=NUM_CORES
    )

    @functools.partial(
        pl.kernel,
        mesh=mesh,
        out_type=jax.ShapeDtypeStruct((b, d), jnp.float32),
        scratch_types=[
            pltpu.VMEM((b_per_w,), jnp.int32),
        ] + [pltpu.VMEM((GATHER_CHUNK, d), jnp.float32)] * NBUF
          + [pltpu.SemaphoreType.DMA] * (2 * NBUF),
    )
    def gather_kernel(table_hbm, idx_hbm, out_hbm, idx_v, *bufs_and_sems):
        bufs = bufs_and_sems[:NBUF]
        gsems = bufs_and_sems[NBUF:2 * NBUF]
        wsems = bufs_and_sems[2 * NBUF:]
        wid = lax.axis_index("c") * NUM_SUBCORES + lax.axis_index("s")
        base = wid * b_per_w
        pltpu.sync_copy(idx_hbm.at[pl.ds(base, b_per_w)], idx_v)

        def issue_gather(i):
            pltpu.async_copy(
                table_hbm.at[idx_v.at[pl.ds(i * GATHER_CHUNK, GATHER_CHUNK)]],
                bufs[i % NBUF], gsems[i % NBUF],
            )

        for i in range(min(AHEAD, n_chunks)):
            issue_gather(i)
        for i in range(n_chunks):
            j = i + AHEAD
            if j < n_chunks:
                if i >= 1:
                    pltpu.make_async_copy(
                        bufs[(i - 1) % NBUF],
                        out_hbm.at[pl.ds(0, GATHER_CHUNK)],
                        wsems[(i - 1) % NBUF],
                    ).wait()
                issue_gather(j)
            pltpu.make_async_copy(
                table_hbm.at[pl.ds(0, GATHER_CHUNK)], bufs[i % NBUF],
                gsems[i % NBUF],
            ).wait()
            pltpu.async_copy(
                bufs[i % NBUF],
                out_hbm.at[pl.ds(base + i * GATHER_CHUNK, GATHER_CHUNK)],
                wsems[i % NBUF],
            )
        for i in range(max(0, n_chunks - NBUF), n_chunks):
            pltpu.make_async_copy(
                bufs[i % NBUF], out_hbm.at[pl.ds(0, GATHER_CHUNK)],
                wsems[i % NBUF],
            ).wait()

    return gather_kernel(table, idx)


def _ln_body(x_ref, pos_ref, gamma_ref, beta_ref, *rest):
    out_ref = rest[-1]
    pos_start = (pl.program_id(0) % (pos_ref.shape[0] // x_ref.shape[0])) \
        * x_ref.shape[0]
    x = x_ref[...] + pos_ref[pl.ds(pos_start, x_ref.shape[0]), :]
    mean = jnp.mean(x, axis=-1, keepdims=True)
    centered = x - mean
    var = jnp.mean(centered * centered, axis=-1, keepdims=True)
    normed = centered * lax.rsqrt(var + EPS)
    out_ref[...] = normed * gamma_ref[...] + beta_ref[...]


def _tc_add_ln_part(gathered, pos_table, gamma, beta, part_idx, n_total,
                    block_rows, prev_out):
    rows, d = gathered.shape
    s = pos_table.shape[0]
    blocks_per_part = rows // block_rows
    in_specs = [
        pl.BlockSpec((block_rows, d), lambda i: (i, 0)),
        pl.BlockSpec((s, d), lambda i: (0, 0)),
        pl.BlockSpec((1, d), lambda i: (0, 0)),
        pl.BlockSpec((1, d), lambda i: (0, 0)),
    ]
    operands = [gathered, pos_table, gamma.reshape(1, d), beta.reshape(1, d)]
    aliases = {}
    if prev_out is not None:
        in_specs.append(pl.BlockSpec(memory_space=pl.ANY))
        operands.append(prev_out)
        aliases = {4: 0}
    base = part_idx * blocks_per_part
    return pl.pallas_call(
        _ln_body,
        grid=(blocks_per_part,),
        in_specs=in_specs,
        out_specs=pl.BlockSpec((block_rows, d), lambda i: (base + i, 0)),
        out_shape=jax.ShapeDtypeStruct((n_total, d), jnp.float32),
        input_output_aliases=aliases,
        compiler_params=pltpu.CompilerParams(
            dimension_semantics=("arbitrary",),
        ),
    )(*operands)


NSPLIT = 1


def kernel(input_ids, word_table, pos_table, gamma, beta):
    batch, seq = input_ids.shape
    d = word_table.shape[1]
    n = batch * seq
    ids_flat = input_ids.reshape(-1).astype(jnp.int32)
    rows_per_split = n // NSPLIT
    gathered = [
        _sc_gather(
            word_table,
            lax.slice(ids_flat, (h * rows_per_split,),
                      ((h + 1) * rows_per_split,)),
        )
        for h in range(NSPLIT)
    ]
    out = None
    for h in range(NSPLIT):
        out = _tc_add_ln_part(gathered[h], pos_table, gamma, beta, h, n,
                              block_rows=1024, prev_out=out)
    return out.reshape(batch, seq, d)


# speedup vs baseline: 1.0111x; 1.0111x over previous
import functools

import jax
import jax.numpy as jnp
from jax import lax
from jax.experimental import pallas as pl
from jax.experimental.pallas import tpu as pltpu
from jax.experimental.pallas import tpu_sc as plsc

EPS = 1e-12

NUM_CORES = 2
NUM_SUBCORES = 16
GATHER_CHUNK = 32
NBUF = 4
AHEAD = 3


def _sc_gather(table, idx):
    b, = idx.shape
    _, d = table.shape
    n_workers = NUM_CORES * NUM_SUBCORES
    b_per_w = b // n_workers
    n_chunks = b_per_w // GATHER_CHUNK
    mesh = plsc.VectorSubcoreMesh(
        core_axis_name="c", subcore_axis_name="s", num_cores=NUM_CORES
    )

    @functools.partial(
        pl.kernel,
        mesh=mesh,
        out_type=jax.ShapeDtypeStruct((b, d), jnp.float32),
        scratch_types=[
            pltpu.VMEM((b_per_w,), jnp.int32),
        ] + [pltpu.VMEM((GATHER_CHUNK, d), jnp.float32)] * NBUF
          + [pltpu.SemaphoreType.DMA] * (2 * NBUF),
    )
    def gather_kernel(table_hbm, idx_hbm, out_hbm, idx_v, *bufs_and_sems):
        bufs = bufs_and_sems[:NBUF]
        gsems = bufs_and_sems[NBUF:2 * NBUF]
        wsems = bufs_and_sems[2 * NBUF:]
        wid = lax.axis_index("c") * NUM_SUBCORES + lax.axis_index("s")
        base = wid * b_per_w
        pltpu.sync_copy(idx_hbm.at[pl.ds(base, b_per_w)], idx_v)

        def issue_gather(i):
            pltpu.async_copy(
                table_hbm.at[idx_v.at[pl.ds(i * GATHER_CHUNK, GATHER_CHUNK)]],
                bufs[i % NBUF], gsems[i % NBUF],
            )

        for i in range(min(AHEAD, n_chunks)):
            issue_gather(i)
        for i in range(n_chunks):
            j = i + AHEAD
            if j < n_chunks:
                if i >= 1:
                    pltpu.make_async_copy(
                        bufs[(i - 1) % NBUF],
                        out_hbm.at[pl.ds(0, GATHER_CHUNK)],
                        wsems[(i - 1) % NBUF],
                    ).wait()
                issue_gather(j)
            pltpu.make_async_copy(
                table_hbm.at[pl.ds(0, GATHER_CHUNK)], bufs[i % NBUF],
                gsems[i % NBUF],
            ).wait()
            pltpu.async_copy(
                bufs[i % NBUF],
                out_hbm.at[pl.ds(base + i * GATHER_CHUNK, GATHER_CHUNK)],
                wsems[i % NBUF],
            )
        for i in range(max(0, n_chunks - NBUF), n_chunks):
            pltpu.make_async_copy(
                bufs[i % NBUF], out_hbm.at[pl.ds(0, GATHER_CHUNK)],
                wsems[i % NBUF],
            ).wait()

    return gather_kernel(table, idx)


def _ln_body(x_ref, pos_ref, gamma_ref, beta_ref, *rest):
    out_ref = rest[-1]
    pos_start = (pl.program_id(0) % (pos_ref.shape[0] // x_ref.shape[0])) \
        * x_ref.shape[0]
    x = x_ref[...] + pos_ref[pl.ds(pos_start, x_ref.shape[0]), :]
    mean = jnp.mean(x, axis=-1, keepdims=True)
    centered = x - mean
    var = jnp.mean(centered * centered, axis=-1, keepdims=True)
    normed = centered * lax.rsqrt(var + EPS)
    out_ref[...] = normed * gamma_ref[...] + beta_ref[...]


def _tc_add_ln_part(gathered, pos_table, gamma, beta, part_idx, n_total,
                    block_rows, prev_out):
    rows, d = gathered.shape
    s = pos_table.shape[0]
    blocks_per_part = rows // block_rows
    in_specs = [
        pl.BlockSpec((block_rows, d), lambda i: (i, 0)),
        pl.BlockSpec((s, d), lambda i: (0, 0)),
        pl.BlockSpec((1, d), lambda i: (0, 0)),
        pl.BlockSpec((1, d), lambda i: (0, 0)),
    ]
    operands = [gathered, pos_table, gamma.reshape(1, d), beta.reshape(1, d)]
    aliases = {}
    if prev_out is not None:
        in_specs.append(pl.BlockSpec(memory_space=pl.ANY))
        operands.append(prev_out)
        aliases = {4: 0}
    base = part_idx * blocks_per_part
    return pl.pallas_call(
        _ln_body,
        grid=(blocks_per_part,),
        in_specs=in_specs,
        out_specs=pl.BlockSpec((block_rows, d), lambda i: (base + i, 0)),
        out_shape=jax.ShapeDtypeStruct((n_total, d), jnp.float32),
        input_output_aliases=aliases,
        compiler_params=pltpu.CompilerParams(
            dimension_semantics=("arbitrary",),
        ),
    )(*operands)


NSPLIT = 1


def kernel(input_ids, word_table, pos_table, gamma, beta):
    batch, seq = input_ids.shape
    d = word_table.shape[1]
    n = batch * seq
    ids_flat = input_ids.reshape(-1).astype(jnp.int32)
    rows_per_split = n // NSPLIT
    gathered = [
        _sc_gather(
            word_table,
            lax.slice(ids_flat, (h * rows_per_split,),
                      ((h + 1) * rows_per_split,)),
        )
        for h in range(NSPLIT)
    ]
    out = None
    for h in range(NSPLIT):
        out = _tc_add_ln_part(gathered[h], pos_table, gamma, beta, h, n,
                              block_rows=2048, prev_out=out)
    return out.reshape(batch, seq, d)
